# baseline (speedup 1.0000x reference)
import functools

import jax
import jax.numpy as jnp
from jax import lax
from jax.experimental import pallas as pl
from jax.experimental.pallas import tpu as pltpu
from jax.experimental.pallas import tpu_sc as plsc

EMB = 128
K_TOP = 2048
LANES = 16
NC = 2
NS = 16
NW = NC * NS
CHUNK = 128
ACHUNK = 128
NA = 10240
USE_SC_AGG = True
USE_SC_SCORE = True
AGG_STAGE = 4


def _k1_body(hb, tp, wt, rp, rw1, rb1, rw2, rb2, wr0, wr1, wq, qe, pb1, wrb,
             h0_o, p0_o, p1_o, zr_o, c_o):
    f32 = jnp.float32
    h0_o[...] = hb[...] + tp[...] * wt[...]
    r = rp[...]
    rrev = jnp.maximum(jnp.dot(r, rw1[...], preferred_element_type=f32) + rb1[...], 0.0)
    rrev = jnp.dot(rrev, rw2[...], preferred_element_type=f32) + rb2[...]
    p0_o[0:512, :] = jnp.dot(r, wr0[...], preferred_element_type=f32)
    p0_o[512:1024, :] = jnp.dot(rrev, wr0[...], preferred_element_type=f32)
    p1_o[0:512, :] = jnp.dot(r, wr1[...], preferred_element_type=f32)
    p1_o[512:1024, :] = jnp.dot(rrev, wr1[...], preferred_element_type=f32)
    zr_o[...] = jnp.dot(r, wrb[...], preferred_element_type=f32)
    c_o[...] = jnp.dot(qe[...], wq[...], preferred_element_type=f32) + pb1[...]


def _k1(hb, tp, wt, rp, rw1, rb1, rw2, rb2, wr0, wr1, wq, qe, pb1, wrb):
    f32 = jnp.float32
    return pl.pallas_call(
        _k1_body,
        out_shape=(
            jax.ShapeDtypeStruct((10000, EMB), f32),
            jax.ShapeDtypeStruct((1024, EMB), f32),
            jax.ShapeDtypeStruct((1024, EMB), f32),
            jax.ShapeDtypeStruct((512, EMB), f32),
            jax.ShapeDtypeStruct((1, EMB), f32),
        ),
    )(hb, tp, wt, rp, rw1, rb1, rw2, rb2, wr0, wr1, wq, qe, pb1, wrb)


def _sc_agg_body(mode, e2p, h_hbm, p_hbm, src_hbm, rel_hbm, dst_hbm,
                 part_hbm, acc, src_v, rel_v, dst_v, hrows, prows,
                 sem_h, sem_p):
    f32 = jnp.float32
    i32 = jnp.int32
    c = lax.axis_index("c")
    s = lax.axis_index("s")
    wid = c * NS + s
    rps = NA // NS
    r0 = s * rps
    iota16 = lax.iota(i32, LANES)
    nstripe = rps // ACHUNK

    def fill_idx(off):
        for k in range(ACHUNK // LANES):
            src_v[pl.ds(k * LANES, LANES)] = (
                jnp.full((LANES,), r0 + off + k * LANES, i32) + iota16)

    def fill_rows(ref, val):
        def body(i, carry):
            ref[i // (EMB // LANES),
                pl.ds((i % (EMB // LANES)) * LANES, LANES)] = jnp.full(
                    (LANES,), val, f32)
            return carry
        lax.fori_loop(0, ACHUNK * (EMB // LANES), body, 0)

    fill_rows(hrows, 0.0)
    for t in range(nstripe):
        fill_idx(t * ACHUNK)
        pltpu.sync_copy(hrows, acc.at[src_v])
    if mode == "deg":
        fill_rows(hrows, 1.0)
    plsc.subcore_barrier()

    per_w = e2p // NW
    nchunk = per_w // ACHUNK
    base_w = wid * per_w

    def chunk(j, carry):
        b = base_w + j * ACHUNK
        pltpu.sync_copy(dst_hbm.at[pl.ds(b, ACHUNK)], dst_v)
        if mode == "agg":
            pltpu.sync_copy(src_hbm.at[pl.ds(b, ACHUNK)], src_v)
            pltpu.sync_copy(rel_hbm.at[pl.ds(b, ACHUNK)], rel_v)
            cp1 = pltpu.async_copy(h_hbm.at[src_v], hrows, sem_h)
            cp2 = pltpu.async_copy(p_hbm.at[rel_v], prows, sem_p)
            cp1.wait()
            cp2.wait()

            def addp(i, carry2):
                e = i // (EMB // LANES)
                sl = pl.ds((i % (EMB // LANES)) * LANES, LANES)
                hrows[e, sl] = hrows[e, sl] + prows[e, sl]
                return carry2

            lax.fori_loop(0, ACHUNK * (EMB // LANES), addp, 0)
            pltpu.sync_copy(hrows, acc.at[dst_v], add=True)
        else:
            pltpu.sync_copy(hrows, acc.at[dst_v], add=True)
        return carry

    lax.fori_loop(0, nchunk, chunk, 0)
    plsc.subcore_barrier()
    for t in range(nstripe):
        off = t * ACHUNK
        fill_idx(off)
        pltpu.async_copy(acc.at[src_v], hrows, sem_h).wait()
        pltpu.sync_copy(hrows, part_hbm.at[c, pl.ds(r0 + off, ACHUNK)])


def _sc_agg(mode, e2p, h_tab, p_tab, src, rel, dst):
    f32 = jnp.float32
    mesh = plsc.VectorSubcoreMesh(core_axis_name="c", subcore_axis_name="s")
    cparams = pltpu.CompilerParams(needs_layout_passes=False)
    return pl.kernel(
        functools.partial(_sc_agg_body, mode, e2p),
        out_type=jax.ShapeDtypeStruct((NC, NA, EMB), f32),
        mesh=mesh,
        scratch_types=[
            pltpu.VMEM_SHARED((NA, EMB), f32),
            pltpu.VMEM((ACHUNK,), jnp.int32),
            pltpu.VMEM((ACHUNK,), jnp.int32),
            pltpu.VMEM((ACHUNK,), jnp.int32),
            pltpu.VMEM((ACHUNK, EMB), f32),
            pltpu.VMEM((ACHUNK, EMB), f32),
            pltpu.SemaphoreType.DMA,
            pltpu.SemaphoreType.DMA,
        ],
        compiler_params=cparams,
    )(h_tab, p_tab, src, rel, dst)


def _k2a_body(h_r, part_r, dp_r, sel_r, ws_r, wn_r, b_r, h1_o):
    f32 = jnp.float32
    deg = jnp.maximum(jnp.dot(dp_r[0] + dp_r[1], sel_r[...],
                              preferred_element_type=f32), 1.0)
    agg = (part_r[0] + part_r[1]) / deg
    h1_o[...] = jnp.maximum(
        jnp.dot(h_r[...], ws_r[...], preferred_element_type=f32)
        + jnp.dot(agg[0:10000, :], wn_r[...], preferred_element_type=f32)
        + b_r[...], 0.0)


def _k2a(h0, part, dp, sel, ws, wn, b):
    return pl.pallas_call(
        _k2a_body,
        out_shape=jax.ShapeDtypeStruct((10000, EMB), jnp.float32),
    )(h0, part, dp, sel, ws, wn, b)


def _k2b_body(h1_r, part_r, dp_r, sel_r, ws_r, wn_r, b_r, h0_r,
              wh0_r, wh1_r, wh2_r, wt0_r, wt1_r, wt2_r, c_r,
              zh_o, zt_o):
    f32 = jnp.float32
    deg = jnp.maximum(jnp.dot(dp_r[0] + dp_r[1], sel_r[...],
                              preferred_element_type=f32), 1.0)
    agg = (part_r[0] + part_r[1]) / deg
    h2 = jnp.maximum(
        jnp.dot(h1_r[...], ws_r[...], preferred_element_type=f32)
        + jnp.dot(agg[0:10000, :], wn_r[...], preferred_element_type=f32)
        + b_r[...], 0.0)
    zh_o[...] = (jnp.dot(h0_r[...], wh0_r[...], preferred_element_type=f32)
                 + jnp.dot(h1_r[...], wh1_r[...], preferred_element_type=f32)
                 + jnp.dot(h2, wh2_r[...], preferred_element_type=f32)
                 + c_r[...])
    zt_o[...] = (jnp.dot(h0_r[...], wt0_r[...], preferred_element_type=f32)
                 + jnp.dot(h1_r[...], wt1_r[...], preferred_element_type=f32)
                 + jnp.dot(h2, wt2_r[...], preferred_element_type=f32))


def _k2b(h1, part, dp, sel, ws, wn, b, h0, wh0, wh1, wh2, wt0, wt1, wt2, cvec):
    f32 = jnp.float32
    return pl.pallas_call(
        _k2b_body,
        out_shape=(jax.ShapeDtypeStruct((10000, EMB), f32),
                   jax.ShapeDtypeStruct((10000, EMB), f32)),
    )(h1, part, dp, sel, ws, wn, b, h0, wh0, wh1, wh2, wt0, wt1, wt2, cvec)


def _sc_score_body(ep, zh_hbm, zt_hbm, zr_hbm, hid_hbm, rid_hbm, tid_hbm,
                   v_hbm, hid_v, rid_v, tid_v,
                   arows, brows, crows, sem_a, sem_b, sem_c):
    c = lax.axis_index("c")
    s = lax.axis_index("s")
    wid = c * NS + s
    per_w = ep // NW
    nchunk = per_w // CHUNK
    base_w = wid * per_w

    def chunk(j, carry):
        b = base_w + j * CHUNK
        pltpu.sync_copy(hid_hbm.at[pl.ds(b, CHUNK)], hid_v)
        pltpu.sync_copy(rid_hbm.at[pl.ds(b, CHUNK)], rid_v)
        pltpu.sync_copy(tid_hbm.at[pl.ds(b, CHUNK)], tid_v)
        cpa = pltpu.async_copy(zh_hbm.at[hid_v], arows, sem_a)
        cpb = pltpu.async_copy(zr_hbm.at[rid_v], brows, sem_b)
        cpc = pltpu.async_copy(zt_hbm.at[tid_v], crows, sem_c)
        cpa.wait()
        cpb.wait()
        cpc.wait()

        def accvec(i, carry2):
            e = i // (EMB // LANES)
            sl = pl.ds((i % (EMB // LANES)) * LANES, LANES)
            arows[e, sl] = arows[e, sl] + brows[e, sl] + crows[e, sl]
            return carry2

        lax.fori_loop(0, CHUNK * (EMB // LANES), accvec, 0)
        pltpu.sync_copy(arows, v_hbm.at[pl.ds(b, CHUNK)])
        return carry

    lax.fori_loop(0, nchunk, chunk, 0)


def _sc_score(ep, zh, zt, zr, hid, rid, tid):
    f32 = jnp.float32
    mesh = plsc.VectorSubcoreMesh(core_axis_name="c", subcore_axis_name="s")
    return pl.kernel(
        functools.partial(_sc_score_body, ep),
        out_type=jax.ShapeDtypeStruct((ep, EMB), f32),
        mesh=mesh,
        compiler_params=pltpu.CompilerParams(needs_layout_passes=False),
        scratch_types=[
            pltpu.VMEM((CHUNK,), jnp.int32),
            pltpu.VMEM((CHUNK,), jnp.int32),
            pltpu.VMEM((CHUNK,), jnp.int32),
            pltpu.VMEM((CHUNK, EMB), f32),
            pltpu.VMEM((CHUNK, EMB), f32),
            pltpu.VMEM((CHUNK, EMB), f32),
            pltpu.SemaphoreType.DMA,
            pltpu.SemaphoreType.DMA,
            pltpu.SemaphoreType.DMA,
        ],
    )(zh, zt, zr, hid, rid, tid)


def _k3c_body(v_r, w2_r, o_r):
    o_r[...] = jnp.dot(jnp.maximum(v_r[...], 0.0), w2_r[...],
                       preferred_element_type=jnp.float32)


def _k3c(v, w2col):
    ep = v.shape[0]
    blk = 8192
    return pl.pallas_call(
        _k3c_body,
        grid=(ep // blk,),
        in_specs=[
            pl.BlockSpec((blk, EMB), lambda i: (i, 0)),
            pl.BlockSpec((EMB, 1), lambda i: (0, 0)),
        ],
        out_specs=pl.BlockSpec((blk, 1), lambda i: (i, 0)),
        out_shape=jax.ShapeDtypeStruct((ep, 1), jnp.float32),
    )(v, w2col)


def _k4_body(att_r, b2_r, g_r, out_r):
    rows, cols = att_r.shape
    pert = (att_r[...] + b2_r[...]) + g_r[...]
    soft = jax.nn.sigmoid(pert)
    b = lax.bitcast_convert_type(pert, jnp.int32)
    key = jnp.where(b < 0, b ^ jnp.int32(0x7FFFFFFF), b)

    def bit_step(i, lo):
        cand = lo + (jnp.int32(1) << (jnp.int32(30) - i))
        cnt = jnp.sum((key >= cand).astype(jnp.int32))
        return jnp.where(cnt >= K_TOP, cand, lo)

    cnt0 = jnp.sum((key >= 0).astype(jnp.int32))
    lo_init = jnp.where(cnt0 >= K_TOP, jnp.int32(0), jnp.int32(-2147483648))
    t = lax.fori_loop(0, 31, bit_step, lo_init)
    cnt_gt = jnp.sum((key > t).astype(jnp.int32))
    need = K_TOP - cnt_gt
    eq = key == t
    idx = (lax.broadcasted_iota(jnp.int32, (rows, cols), 0) * cols
           + lax.broadcasted_iota(jnp.int32, (rows, cols), 1))

    def idx_step(i, lo2):
        cand = lo2 + (jnp.int32(1) << (jnp.int32(17) - i))
        cnt = jnp.sum((eq & (idx < cand)).astype(jnp.int32))
        return jnp.where(cnt <= need, cand, lo2)

    isel = lax.fori_loop(0, 18, idx_step, jnp.int32(0))
    hard = ((key > t) | (eq & (idx < isel))).astype(jnp.float32)
    out_r[...] = (hard + soft) - soft


def _k4(att2d, b2row, g2d):
    return pl.pallas_call(
        _k4_body,
        out_shape=jax.ShapeDtypeStruct(att2d.shape, jnp.float32),
    )(att2d, b2row, g2d)


def kernel(h_id_tensor, r_id_tensor, t_id_tensor, q_emb, entity_embs,
           num_non_text_entities, relation_embs, topic_entity_one_hot, dummy,
           non_text_emb, W_topic, sage_Wself, sage_Wneigh, sage_Wrel, sage_b,
           rev_W1, rev_b1, rev_W2, rev_b2, pred_W1, pred_b1, pred_W2, pred_b2):
    f32 = jnp.float32
    E = h_id_tensor.shape[0]
    N = topic_entity_one_hot.shape[0]
    n_text = entity_embs.shape[0]
    R = relation_embs.shape[0]
    num_non_text_static = N - n_text
    C3 = (sage_Wself.shape[0] + 1) * EMB

    zero_dep = (jnp.asarray(num_non_text_entities) - num_non_text_static).astype(f32)
    h_base = jnp.concatenate(
        [entity_embs, jnp.broadcast_to(non_text_emb, (num_non_text_static, EMB))],
        axis=0) + zero_dep
    rpad = jnp.zeros((512, EMB), f32).at[:R].set(relation_embs)
    b1row = rev_b1.reshape(1, EMB)
    b2row = rev_b2.reshape(1, EMB)
    pb1row = pred_b1.reshape(1, EMB)
    wq = pred_W1[0:EMB]
    wh = [pred_W1[EMB + l * EMB: EMB + (l + 1) * EMB] for l in range(3)]
    wrb = pred_W1[EMB + C3: 2 * EMB + C3]
    wt = [pred_W1[2 * EMB + C3 + l * EMB: 2 * EMB + C3 + (l + 1) * EMB]
          for l in range(3)]
    sageb = [sage_b[l].reshape(1, EMB) for l in range(2)]

    h0, p0, p1, zr, cvec = _k1(
        h_base, topic_entity_one_hot.astype(f32), W_topic, rpad,
        rev_W1, b1row, rev_W2, b2row, sage_Wrel[0], sage_Wrel[1],
        wq, q_emb, pb1row, wrb)

    hid = h_id_tensor.astype(jnp.int32)
    rid = r_id_tensor.astype(jnp.int32)
    tid = t_id_tensor.astype(jnp.int32)
    e2 = 2 * E
    per_w2 = ((e2 // NW + ACHUNK - 1) // ACHUNK) * ACHUNK
    e2p = per_w2 * NW
    src = jnp.concatenate([hid, tid])
    dst = jnp.concatenate([tid, hid])
    rel = jnp.concatenate([rid, rid + 512])
    src = jnp.pad(src, (0, e2p - e2))
    rel = jnp.pad(rel, (0, e2p - e2))
    dst = jnp.pad(dst, (0, e2p - e2), constant_values=N)
    sel = (lax.broadcasted_iota(jnp.int32, (EMB, EMB), 0) == 0).astype(f32)

    dp = _sc_agg("deg", e2p, h0, p0, src, rel, dst)
    part0 = _sc_agg("agg", e2p, h0, p0, src, rel, dst)
    h1 = _k2a(h0, part0, dp, sel, sage_Wself[0], sage_Wneigh[0], sageb[0])

    part1 = _sc_agg("agg", e2p, h1, p1, src, rel, dst)
    zh, zt = _k2b(h1, part1, dp, sel, sage_Wself[1], sage_Wneigh[1], sageb[1],
                  h0, wh[0], wh[1], wh[2], wt[0], wt[1], wt[2], cvec)

    per_w = ((E // NW + CHUNK - 1) // CHUNK) * CHUNK
    ep = per_w * NW
    hidp = jnp.pad(hid, (0, ep - E))
    ridp = jnp.pad(rid, (0, ep - E))
    tidp = jnp.pad(tid, (0, ep - E))
    vrows = _sc_score(ep, zh, zt, zr, hidp, ridp, tidp)
    att_raw = _k3c(vrows, pred_W2)

    att2d = att_raw[:E, 0].reshape(E // EMB, EMB)
    u = jax.random.uniform(jax.random.key(42), (E, 1), minval=1e-10,
                           maxval=1.0 - 1e-10)
    g2d = (-jnp.log(-jnp.log(u))).reshape(E // EMB, EMB)
    b2b = jnp.broadcast_to(pred_b2.reshape(1, 1), (1, EMB))
    out2d = _k4(att2d, b2b, g2d)
    return out2d.reshape(E, 1)


# speedup vs baseline: 1.3142x; 1.3142x over previous
import functools

import jax
import jax.numpy as jnp
from jax import lax
from jax.experimental import pallas as pl
from jax.experimental.pallas import tpu as pltpu
from jax.experimental.pallas import tpu_sc as plsc

EMB = 128
K_TOP = 2048
LANES = 16
NC = 2
NS = 16
NW = NC * NS
CHUNK = 128
ACHUNK = 128
NA = 10240
USE_SC_AGG = True
USE_SC_SCORE = True
AGG_STAGE = 4


def _k1_body(hb, tp, wt, rp, rw1, rb1, rw2, rb2, wr0, wr1, wq, qe, pb1, wrb,
             h0_o, p0_o, p1_o, zr_o, c_o):
    f32 = jnp.float32
    h0_o[...] = hb[...] + tp[...] * wt[...]
    r = rp[...]
    rrev = jnp.maximum(jnp.dot(r, rw1[...], preferred_element_type=f32) + rb1[...], 0.0)
    rrev = jnp.dot(rrev, rw2[...], preferred_element_type=f32) + rb2[...]
    p0_o[0:512, :] = jnp.dot(r, wr0[...], preferred_element_type=f32)
    p0_o[512:1024, :] = jnp.dot(rrev, wr0[...], preferred_element_type=f32)
    p1_o[0:512, :] = jnp.dot(r, wr1[...], preferred_element_type=f32)
    p1_o[512:1024, :] = jnp.dot(rrev, wr1[...], preferred_element_type=f32)
    zr_o[...] = jnp.dot(r, wrb[...], preferred_element_type=f32)
    c_o[...] = jnp.dot(qe[...], wq[...], preferred_element_type=f32) + pb1[...]


def _k1(hb, tp, wt, rp, rw1, rb1, rw2, rb2, wr0, wr1, wq, qe, pb1, wrb):
    f32 = jnp.float32
    return pl.pallas_call(
        _k1_body,
        out_shape=(
            jax.ShapeDtypeStruct((10000, EMB), f32),
            jax.ShapeDtypeStruct((1024, EMB), f32),
            jax.ShapeDtypeStruct((1024, EMB), f32),
            jax.ShapeDtypeStruct((512, EMB), f32),
            jax.ShapeDtypeStruct((1, EMB), f32),
        ),
    )(hb, tp, wt, rp, rw1, rb1, rw2, rb2, wr0, wr1, wq, qe, pb1, wrb)


def _sc_agg_body(mode, e2p, h_hbm, p_hbm, src_hbm, rel_hbm, dst_hbm,
                 part_hbm, acc, src_v, rel_v, dst_v, hrows, prows,
                 sem_h, sem_p):
    f32 = jnp.float32
    i32 = jnp.int32
    c = lax.axis_index("c")
    s = lax.axis_index("s")
    wid = c * NS + s
    rps = NA // NS
    r0 = s * rps
    iota16 = lax.iota(i32, LANES)
    nstripe = rps // ACHUNK

    def fill_idx(off):
        for k in range(ACHUNK // LANES):
            src_v[pl.ds(k * LANES, LANES)] = (
                jnp.full((LANES,), r0 + off + k * LANES, i32) + iota16)

    def fill_rows(ref, val):
        def body(i, carry):
            ref[i // (EMB // LANES),
                pl.ds((i % (EMB // LANES)) * LANES, LANES)] = jnp.full(
                    (LANES,), val, f32)
            return carry
        lax.fori_loop(0, ACHUNK * (EMB // LANES), body, 0)

    fill_rows(hrows, 0.0)
    for t in range(nstripe):
        fill_idx(t * ACHUNK)
        pltpu.sync_copy(hrows, acc.at[src_v])
    if mode == "deg":
        fill_rows(hrows, 1.0)
    plsc.subcore_barrier()

    per_w = e2p // NW
    nchunk = per_w // ACHUNK
    base_w = wid * per_w

    def chunk(j, carry):
        b = base_w + j * ACHUNK
        pltpu.sync_copy(dst_hbm.at[pl.ds(b, ACHUNK)], dst_v)
        if mode == "agg":
            pltpu.sync_copy(src_hbm.at[pl.ds(b, ACHUNK)], src_v)
            pltpu.sync_copy(rel_hbm.at[pl.ds(b, ACHUNK)], rel_v)
            cp1 = pltpu.async_copy(h_hbm.at[src_v], hrows, sem_h)
            cp2 = pltpu.async_copy(p_hbm.at[rel_v], prows, sem_p)
            cp1.wait()
            cp2.wait()
            pltpu.sync_copy(hrows, acc.at[dst_v], add=True)
            pltpu.sync_copy(prows, acc.at[dst_v], add=True)
        else:
            pltpu.sync_copy(hrows, acc.at[dst_v], add=True)
        return carry

    lax.fori_loop(0, nchunk, chunk, 0)
    plsc.subcore_barrier()
    for t in range(nstripe):
        off = t * ACHUNK
        fill_idx(off)
        pltpu.async_copy(acc.at[src_v], hrows, sem_h).wait()
        pltpu.sync_copy(hrows, part_hbm.at[c, pl.ds(r0 + off, ACHUNK)])


def _sc_agg(mode, e2p, h_tab, p_tab, src, rel, dst):
    f32 = jnp.float32
    mesh = plsc.VectorSubcoreMesh(core_axis_name="c", subcore_axis_name="s")
    cparams = pltpu.CompilerParams(needs_layout_passes=False)
    return pl.kernel(
        functools.partial(_sc_agg_body, mode, e2p),
        out_type=jax.ShapeDtypeStruct((NC, NA, EMB), f32),
        mesh=mesh,
        scratch_types=[
            pltpu.VMEM_SHARED((NA, EMB), f32),
            pltpu.VMEM((ACHUNK,), jnp.int32),
            pltpu.VMEM((ACHUNK,), jnp.int32),
            pltpu.VMEM((ACHUNK,), jnp.int32),
            pltpu.VMEM((ACHUNK, EMB), f32),
            pltpu.VMEM((ACHUNK, EMB), f32),
            pltpu.SemaphoreType.DMA,
            pltpu.SemaphoreType.DMA,
        ],
        compiler_params=cparams,
    )(h_tab, p_tab, src, rel, dst)


def _k2a_body(h_r, part_r, dp_r, sel_r, ws_r, wn_r, b_r, h1_o):
    f32 = jnp.float32
    deg = jnp.maximum(jnp.dot(dp_r[0] + dp_r[1], sel_r[...],
                              preferred_element_type=f32), 1.0)
    agg = (part_r[0] + part_r[1]) / deg
    h1_o[...] = jnp.maximum(
        jnp.dot(h_r[...], ws_r[...], preferred_element_type=f32)
        + jnp.dot(agg[0:10000, :], wn_r[...], preferred_element_type=f32)
        + b_r[...], 0.0)


def _k2a(h0, part, dp, sel, ws, wn, b):
    return pl.pallas_call(
        _k2a_body,
        out_shape=jax.ShapeDtypeStruct((10000, EMB), jnp.float32),
    )(h0, part, dp, sel, ws, wn, b)


def _k2b_body(h1_r, part_r, dp_r, sel_r, ws_r, wn_r, b_r, h0_r,
              wh0_r, wh1_r, wh2_r, wt0_r, wt1_r, wt2_r, c_r,
              zh_o, zt_o):
    f32 = jnp.float32
    deg = jnp.maximum(jnp.dot(dp_r[0] + dp_r[1], sel_r[...],
                              preferred_element_type=f32), 1.0)
    agg = (part_r[0] + part_r[1]) / deg
    h2 = jnp.maximum(
        jnp.dot(h1_r[...], ws_r[...], preferred_element_type=f32)
        + jnp.dot(agg[0:10000, :], wn_r[...], preferred_element_type=f32)
        + b_r[...], 0.0)
    zh_o[...] = (jnp.dot(h0_r[...], wh0_r[...], preferred_element_type=f32)
                 + jnp.dot(h1_r[...], wh1_r[...], preferred_element_type=f32)
                 + jnp.dot(h2, wh2_r[...], preferred_element_type=f32)
                 + c_r[...])
    zt_o[...] = (jnp.dot(h0_r[...], wt0_r[...], preferred_element_type=f32)
                 + jnp.dot(h1_r[...], wt1_r[...], preferred_element_type=f32)
                 + jnp.dot(h2, wt2_r[...], preferred_element_type=f32))


def _k2b(h1, part, dp, sel, ws, wn, b, h0, wh0, wh1, wh2, wt0, wt1, wt2, cvec):
    f32 = jnp.float32
    return pl.pallas_call(
        _k2b_body,
        out_shape=(jax.ShapeDtypeStruct((10000, EMB), f32),
                   jax.ShapeDtypeStruct((10000, EMB), f32)),
    )(h1, part, dp, sel, ws, wn, b, h0, wh0, wh1, wh2, wt0, wt1, wt2, cvec)


def _sc_score_body(ep, zh_hbm, zt_hbm, zr_hbm, hid_hbm, rid_hbm, tid_hbm,
                   v_hbm, hid_v, rid_v, tid_v,
                   arows, brows, crows, sem_a, sem_b, sem_c):
    c = lax.axis_index("c")
    s = lax.axis_index("s")
    wid = c * NS + s
    per_w = ep // NW
    nchunk = per_w // CHUNK
    base_w = wid * per_w

    def chunk(j, carry):
        b = base_w + j * CHUNK
        pltpu.sync_copy(hid_hbm.at[pl.ds(b, CHUNK)], hid_v)
        pltpu.sync_copy(rid_hbm.at[pl.ds(b, CHUNK)], rid_v)
        pltpu.sync_copy(tid_hbm.at[pl.ds(b, CHUNK)], tid_v)
        cpa = pltpu.async_copy(zh_hbm.at[hid_v], arows, sem_a)
        cpb = pltpu.async_copy(zr_hbm.at[rid_v], brows, sem_b)
        cpc = pltpu.async_copy(zt_hbm.at[tid_v], crows, sem_c)
        cpa.wait()
        cpb.wait()
        cpc.wait()

        def accvec(i, carry2):
            e = i // (EMB // LANES)
            sl = pl.ds((i % (EMB // LANES)) * LANES, LANES)
            arows[e, sl] = arows[e, sl] + brows[e, sl] + crows[e, sl]
            return carry2

        lax.fori_loop(0, CHUNK * (EMB // LANES), accvec, 0)
        pltpu.sync_copy(arows, v_hbm.at[pl.ds(b, CHUNK)])
        return carry

    lax.fori_loop(0, nchunk, chunk, 0)


def _sc_score(ep, zh, zt, zr, hid, rid, tid):
    f32 = jnp.float32
    mesh = plsc.VectorSubcoreMesh(core_axis_name="c", subcore_axis_name="s")
    return pl.kernel(
        functools.partial(_sc_score_body, ep),
        out_type=jax.ShapeDtypeStruct((ep, EMB), f32),
        mesh=mesh,
        compiler_params=pltpu.CompilerParams(needs_layout_passes=False),
        scratch_types=[
            pltpu.VMEM((CHUNK,), jnp.int32),
            pltpu.VMEM((CHUNK,), jnp.int32),
            pltpu.VMEM((CHUNK,), jnp.int32),
            pltpu.VMEM((CHUNK, EMB), f32),
            pltpu.VMEM((CHUNK, EMB), f32),
            pltpu.VMEM((CHUNK, EMB), f32),
            pltpu.SemaphoreType.DMA,
            pltpu.SemaphoreType.DMA,
            pltpu.SemaphoreType.DMA,
        ],
    )(zh, zt, zr, hid, rid, tid)


def _k3c_body(v_r, w2_r, o_r):
    o_r[...] = jnp.dot(jnp.maximum(v_r[...], 0.0), w2_r[...],
                       preferred_element_type=jnp.float32)


def _k3c(v, w2col):
    ep = v.shape[0]
    blk = 8192
    return pl.pallas_call(
        _k3c_body,
        grid=(ep // blk,),
        in_specs=[
            pl.BlockSpec((blk, EMB), lambda i: (i, 0)),
            pl.BlockSpec((EMB, 1), lambda i: (0, 0)),
        ],
        out_specs=pl.BlockSpec((blk, 1), lambda i: (i, 0)),
        out_shape=jax.ShapeDtypeStruct((ep, 1), jnp.float32),
    )(v, w2col)


def _k4_body(att_r, b2_r, g_r, out_r):
    rows, cols = att_r.shape
    pert = (att_r[...] + b2_r[...]) + g_r[...]
    soft = jax.nn.sigmoid(pert)
    b = lax.bitcast_convert_type(pert, jnp.int32)
    key = jnp.where(b < 0, b ^ jnp.int32(0x7FFFFFFF), b)

    def bit_step(i, lo):
        cand = lo + (jnp.int32(1) << (jnp.int32(30) - i))
        cnt = jnp.sum((key >= cand).astype(jnp.int32))
        return jnp.where(cnt >= K_TOP, cand, lo)

    cnt0 = jnp.sum((key >= 0).astype(jnp.int32))
    lo_init = jnp.where(cnt0 >= K_TOP, jnp.int32(0), jnp.int32(-2147483648))
    t = lax.fori_loop(0, 31, bit_step, lo_init)
    cnt_gt = jnp.sum((key > t).astype(jnp.int32))
    need = K_TOP - cnt_gt
    eq = key == t
    idx = (lax.broadcasted_iota(jnp.int32, (rows, cols), 0) * cols
           + lax.broadcasted_iota(jnp.int32, (rows, cols), 1))

    def idx_step(i, lo2):
        cand = lo2 + (jnp.int32(1) << (jnp.int32(17) - i))
        cnt = jnp.sum((eq & (idx < cand)).astype(jnp.int32))
        return jnp.where(cnt <= need, cand, lo2)

    isel = lax.fori_loop(0, 18, idx_step, jnp.int32(0))
    hard = ((key > t) | (eq & (idx < isel))).astype(jnp.float32)
    out_r[...] = (hard + soft) - soft


def _k4(att2d, b2row, g2d):
    return pl.pallas_call(
        _k4_body,
        out_shape=jax.ShapeDtypeStruct(att2d.shape, jnp.float32),
    )(att2d, b2row, g2d)


def kernel(h_id_tensor, r_id_tensor, t_id_tensor, q_emb, entity_embs,
           num_non_text_entities, relation_embs, topic_entity_one_hot, dummy,
           non_text_emb, W_topic, sage_Wself, sage_Wneigh, sage_Wrel, sage_b,
           rev_W1, rev_b1, rev_W2, rev_b2, pred_W1, pred_b1, pred_W2, pred_b2):
    f32 = jnp.float32
    E = h_id_tensor.shape[0]
    N = topic_entity_one_hot.shape[0]
    n_text = entity_embs.shape[0]
    R = relation_embs.shape[0]
    num_non_text_static = N - n_text
    C3 = (sage_Wself.shape[0] + 1) * EMB

    zero_dep = (jnp.asarray(num_non_text_entities) - num_non_text_static).astype(f32)
    h_base = jnp.concatenate(
        [entity_embs, jnp.broadcast_to(non_text_emb, (num_non_text_static, EMB))],
        axis=0) + zero_dep
    rpad = jnp.zeros((512, EMB), f32).at[:R].set(relation_embs)
    b1row = rev_b1.reshape(1, EMB)
    b2row = rev_b2.reshape(1, EMB)
    pb1row = pred_b1.reshape(1, EMB)
    wq = pred_W1[0:EMB]
    wh = [pred_W1[EMB + l * EMB: EMB + (l + 1) * EMB] for l in range(3)]
    wrb = pred_W1[EMB + C3: 2 * EMB + C3]
    wt = [pred_W1[2 * EMB + C3 + l * EMB: 2 * EMB + C3 + (l + 1) * EMB]
          for l in range(3)]
    sageb = [sage_b[l].reshape(1, EMB) for l in range(2)]

    h0, p0, p1, zr, cvec = _k1(
        h_base, topic_entity_one_hot.astype(f32), W_topic, rpad,
        rev_W1, b1row, rev_W2, b2row, sage_Wrel[0], sage_Wrel[1],
        wq, q_emb, pb1row, wrb)

    hid = h_id_tensor.astype(jnp.int32)
    rid = r_id_tensor.astype(jnp.int32)
    tid = t_id_tensor.astype(jnp.int32)
    e2 = 2 * E
    per_w2 = ((e2 // NW + ACHUNK - 1) // ACHUNK) * ACHUNK
    e2p = per_w2 * NW
    src = jnp.concatenate([hid, tid])
    dst = jnp.concatenate([tid, hid])
    rel = jnp.concatenate([rid, rid + 512])
    src = jnp.pad(src, (0, e2p - e2))
    rel = jnp.pad(rel, (0, e2p - e2))
    dst = jnp.pad(dst, (0, e2p - e2), constant_values=N)
    sel = (lax.broadcasted_iota(jnp.int32, (EMB, EMB), 0) == 0).astype(f32)

    dp = _sc_agg("deg", e2p, h0, p0, src, rel, dst)
    part0 = _sc_agg("agg", e2p, h0, p0, src, rel, dst)
    h1 = _k2a(h0, part0, dp, sel, sage_Wself[0], sage_Wneigh[0], sageb[0])

    part1 = _sc_agg("agg", e2p, h1, p1, src, rel, dst)
    zh, zt = _k2b(h1, part1, dp, sel, sage_Wself[1], sage_Wneigh[1], sageb[1],
                  h0, wh[0], wh[1], wh[2], wt[0], wt[1], wt[2], cvec)

    per_w = ((E // NW + CHUNK - 1) // CHUNK) * CHUNK
    ep = per_w * NW
    hidp = jnp.pad(hid, (0, ep - E))
    ridp = jnp.pad(rid, (0, ep - E))
    tidp = jnp.pad(tid, (0, ep - E))
    vrows = _sc_score(ep, zh, zt, zr, hidp, ridp, tidp)
    att_raw = _k3c(vrows, pred_W2)

    att2d = att_raw[:E, 0].reshape(E // EMB, EMB)
    u = jax.random.uniform(jax.random.key(42), (E, 1), minval=1e-10,
                           maxval=1.0 - 1e-10)
    g2d = (-jnp.log(-jnp.log(u))).reshape(E // EMB, EMB)
    b2b = jnp.broadcast_to(pred_b2.reshape(1, 1), (1, EMB))
    out2d = _k4(att2d, b2b, g2d)
    return out2d.reshape(E, 1)
